# baseline (speedup 1.0000x reference)
import functools

import jax
import jax.numpy as jnp
from jax import lax
from jax.experimental import pallas as pl
from jax.experimental.pallas import tpu as pltpu
from jax.experimental.pallas import tpu_sc as plsc

H = 768
F = 768
E = 64
TOP_K = 2
T = 8192
P = T * TOP_K
TM = 128
NP = P + E * (TM - 1) + TM - ((P + E * (TM - 1)) % TM or TM)
NP = ((P + E * (TM - 1) + TM - 1) // TM) * TM
NT = NP // TM
NB = P // TM

NC = 2
NS = 16
NW = NC * NS
LANES = 16

_f32 = jnp.float32
_i32 = jnp.int32


def _router_meta_body(x_ref, rw_ref, d0_ref, d1_ref, w1_ref, w2_ref, te_ref):
    x = x_ref[...]
    logits = jnp.dot(x, rw_ref[...], preferred_element_type=_f32)

    iota_e = lax.broadcasted_iota(_i32, (T, E), 1)
    m1 = jnp.max(logits, axis=1, keepdims=True)
    i1 = jnp.min(jnp.where(logits == m1, iota_e, E), axis=1, keepdims=True)
    oh1 = iota_e == i1
    l2 = jnp.where(oh1, -1e30, logits)
    m2 = jnp.max(l2, axis=1, keepdims=True)
    i2 = jnp.min(jnp.where(l2 == m2, iota_e, E), axis=1, keepdims=True)
    oh2 = iota_e == i2

    ex = jnp.exp(m2 - m1)
    w1_ref[...] = 1.0 / (1.0 + ex)
    w2_ref[...] = ex / (1.0 + ex)

    ohw = jnp.concatenate([oh1.astype(_f32), oh2.astype(_f32)], axis=1)

    a = ohw
    s = 1
    while s < T:
        shifted = jnp.concatenate(
            [jnp.zeros((s, TOP_K * E), _f32), a[0:T - s, :]], axis=0)
        a = a + shifted
        s *= 2
    excl = a - ohw

    counts_row = a[T - 1:T, :]
    lane2 = lax.broadcasted_iota(_i32, (1, TOP_K * E), 1)
    mask0 = (lane2 < E).astype(_f32)
    c1 = counts_row[:, 0:E]
    tot = c1 + counts_row[:, E:TOP_K * E]
    pc = ((tot.astype(_i32) + (TM - 1)) >> 7) << 7
    e_r = lax.broadcasted_iota(_i32, (E, E), 0)
    e_c = lax.broadcasted_iota(_i32, (E, E), 1)
    lt_e = (e_r < e_c).astype(_f32)
    offs = jnp.dot(pc.astype(_f32), lt_e, preferred_element_type=_f32)

    addpack = jnp.concatenate([offs, offs + c1], axis=1)
    dm = (excl + addpack) * ohw
    mask0b = jnp.broadcast_to(mask0.astype(jnp.bool_), dm.shape)
    d0 = jnp.sum(jnp.where(mask0b, dm, 0.0), axis=1, keepdims=True)
    d1 = jnp.sum(jnp.where(mask0b, 0.0, dm), axis=1, keepdims=True)
    d0_ref[...] = d0.astype(_i32)
    d1_ref[...] = d1.astype(_i32)

    starts = (lax.broadcasted_iota(_i32, (NT, E), 0) * TM).astype(_f32)
    te = jnp.sum((offs <= starts).astype(_f32), axis=1, keepdims=True) - 1.0
    te_ref[...] = te.astype(_i32)


def _router_meta(x_flat, router_w):
    return pl.pallas_call(
        _router_meta_body,
        grid=(1,),
        in_specs=[
            pl.BlockSpec((T, H), lambda i: (0, 0)),
            pl.BlockSpec((H, E), lambda i: (0, 0)),
        ],
        out_specs=[
            pl.BlockSpec((T, 1), lambda i: (0, 0)),
            pl.BlockSpec((T, 1), lambda i: (0, 0)),
            pl.BlockSpec((T, 1), lambda i: (0, 0)),
            pl.BlockSpec((T, 1), lambda i: (0, 0)),
            pl.BlockSpec((NT, 1), lambda i: (0, 0)),
        ],
        out_shape=[
            jax.ShapeDtypeStruct((T, 1), _i32),
            jax.ShapeDtypeStruct((T, 1), _i32),
            jax.ShapeDtypeStruct((T, 1), _f32),
            jax.ShapeDtypeStruct((T, 1), _f32),
            jax.ShapeDtypeStruct((NT, 1), _i32),
        ],
    )(x_flat, router_w)


_DC = 128
_PPW = P // NW
_DNCH = _PPW // _DC


def _dispatch_body(x_hbm, dest_hbm, w_hbm, xg_hbm, wdest_hbm,
                   tok_v, dest_v, w_v, rows_v, sem):
    wid = lax.axis_index("s") * NC + lax.axis_index("c")
    for ch in range(_DNCH):
        base = wid * _PPW + ch * _DC
        for g in range(_DC // LANES):
            tok_v[pl.ds(g * LANES, LANES)] = jnp.bitwise_and(
                base + g * LANES + lax.iota(_i32, LANES), T - 1)
        pltpu.sync_copy(dest_hbm.at[pl.ds(base, _DC)], dest_v)
        pltpu.sync_copy(w_hbm.at[pl.ds(base, _DC)], w_v)
        pltpu.async_copy(x_hbm.at[tok_v], rows_v, sem).wait()
        pltpu.async_copy(rows_v, xg_hbm.at[dest_v], sem).wait()
        pltpu.async_copy(w_v, wdest_hbm.at[dest_v], sem).wait()


@functools.cache
def _dispatch():
    return pl.kernel(
        _dispatch_body,
        out_type=(
            jax.ShapeDtypeStruct((NP, H), _f32),
            jax.ShapeDtypeStruct((NP,), _f32),
        ),
        mesh=plsc.VectorSubcoreMesh(core_axis_name="c", subcore_axis_name="s",
                                    num_cores=NC, num_subcores=NS),
        scratch_types=[
            pltpu.VMEM((_DC,), _i32),
            pltpu.VMEM((_DC,), _i32),
            pltpu.VMEM((_DC,), _f32),
            pltpu.VMEM((_DC, H), _f32),
            pltpu.SemaphoreType.DMA,
        ],
    )


def _gmm_body(te_ref, xg_ref, wg_ref, wu_ref, wd_ref, ws_ref, y_ref):
    xb = xg_ref[...].astype(jnp.bfloat16)
    g = jnp.dot(xb, wg_ref[0], preferred_element_type=_f32)
    u = jnp.dot(xb, wu_ref[0], preferred_element_type=_f32)
    h = ((g * jax.nn.sigmoid(g)) * u).astype(jnp.bfloat16)
    y = jnp.dot(h, wd_ref[0], preferred_element_type=_f32)
    y_ref[...] = y * ws_ref[...]


def _gmm(te, xg, w_gate, w_up, w_down, wdest):
    grid_spec = pltpu.PrefetchScalarGridSpec(
        num_scalar_prefetch=1,
        grid=(NT,),
        in_specs=[
            pl.BlockSpec((TM, H), lambda m, te_r: (m, 0)),
            pl.BlockSpec((1, H, F), lambda m, te_r: (te_r[m], 0, 0)),
            pl.BlockSpec((1, H, F), lambda m, te_r: (te_r[m], 0, 0)),
            pl.BlockSpec((1, F, H), lambda m, te_r: (te_r[m], 0, 0)),
            pl.BlockSpec((TM, 1), lambda m, te_r: (m, 0)),
        ],
        out_specs=pl.BlockSpec((TM, H), lambda m, te_r: (m, 0)),
    )
    return pl.pallas_call(
        _gmm_body,
        grid_spec=grid_spec,
        out_shape=jax.ShapeDtypeStruct((NP, H), _f32),
    )(te, xg, w_gate, w_up, w_down, wdest)


_CC = 64
_TPW = T // NW
_CNCH = _TPW // _CC


def _combine_body(y_hbm, dest_hbm, out_hbm, d0_v, d1_v, y0_v, y1_v, sem):
    wid = lax.axis_index("s") * NC + lax.axis_index("c")
    for ch in range(_CNCH):
        base = wid * _TPW + ch * _CC
        pltpu.sync_copy(dest_hbm.at[pl.ds(base, _CC)], d0_v)
        pltpu.sync_copy(dest_hbm.at[pl.ds(T + base, _CC)], d1_v)
        pltpu.async_copy(y_hbm.at[d0_v], y0_v, sem).wait()
        pltpu.async_copy(y_hbm.at[d1_v], y1_v, sem).wait()

        def tok(t, _):
            for g in range(H // LANES):
                sl = pl.ds(g * LANES, LANES)
                y0_v[t, sl] = y0_v[t, sl] + y1_v[t, sl]
            return 0

        lax.fori_loop(0, _CC, tok, 0)
        pltpu.sync_copy(y0_v, out_hbm.at[pl.ds(base, _CC)])


@functools.cache
def _combine():
    return pl.kernel(
        _combine_body,
        out_type=jax.ShapeDtypeStruct((T, H), _f32),
        mesh=plsc.VectorSubcoreMesh(core_axis_name="c", subcore_axis_name="s",
                                    num_cores=NC, num_subcores=NS),
        scratch_types=[
            pltpu.VMEM((_CC,), _i32),
            pltpu.VMEM((_CC,), _i32),
            pltpu.VMEM((_CC, H), _f32),
            pltpu.VMEM((_CC, H), _f32),
            pltpu.SemaphoreType.DMA,
        ],
    )


def kernel(x, router_w, w_gate, w_up, w_down):
    b, s, h = x.shape
    x_flat = x.reshape(-1, h)
    d0, d1, w1, w2, te2 = _router_meta(x_flat, router_w)
    dest = jnp.concatenate([d0.reshape(-1), d1.reshape(-1)])
    wflat = jnp.concatenate([w1.reshape(-1), w2.reshape(-1)])
    te = te2.reshape(-1)
    xg, wdest = _dispatch()(x_flat, dest, wflat)
    bf16 = jnp.bfloat16
    y = _gmm(te, xg, w_gate.astype(bf16), w_up.astype(bf16),
             w_down.astype(bf16), wdest.reshape(NP, 1))
    out = _combine()(y, dest)
    return out.reshape(b, s, h)


# speedup vs baseline: 1.2517x; 1.2517x over previous
import functools

import jax
import jax.numpy as jnp
from jax import lax
from jax.experimental import pallas as pl
from jax.experimental.pallas import tpu as pltpu
from jax.experimental.pallas import tpu_sc as plsc

H = 768
F = 768
E = 64
TOP_K = 2
T = 8192
P = T * TOP_K
TM = 128
NP = P + E * (TM - 1) + TM - ((P + E * (TM - 1)) % TM or TM)
NP = ((P + E * (TM - 1) + TM - 1) // TM) * TM
NT = NP // TM
NB = P // TM

NC = 2
NS = 16
NW = NC * NS
LANES = 16

_f32 = jnp.float32
_i32 = jnp.int32


def _router_meta_body(x_ref, rw_ref, d0_ref, d1_ref, w1_ref, w2_ref, te_ref):
    x = x_ref[...]
    logits = jnp.dot(x, rw_ref[...], preferred_element_type=_f32)

    iota_e = lax.broadcasted_iota(_i32, (T, E), 1)
    m1 = jnp.max(logits, axis=1, keepdims=True)
    i1 = jnp.min(jnp.where(logits == m1, iota_e, E), axis=1, keepdims=True)
    oh1 = iota_e == i1
    l2 = jnp.where(oh1, -1e30, logits)
    m2 = jnp.max(l2, axis=1, keepdims=True)
    i2 = jnp.min(jnp.where(l2 == m2, iota_e, E), axis=1, keepdims=True)
    oh2 = iota_e == i2

    ex = jnp.exp(m2 - m1)
    w1_ref[...] = 1.0 / (1.0 + ex)
    w2_ref[...] = ex / (1.0 + ex)

    ohw = jnp.concatenate([oh1.astype(_f32), oh2.astype(_f32)], axis=1)

    a = ohw
    s = 1
    while s < T:
        shifted = jnp.concatenate(
            [jnp.zeros((s, TOP_K * E), _f32), a[0:T - s, :]], axis=0)
        a = a + shifted
        s *= 2
    excl = a - ohw

    counts_row = a[T - 1:T, :]
    lane2 = lax.broadcasted_iota(_i32, (1, TOP_K * E), 1)
    mask0 = (lane2 < E).astype(_f32)
    c1 = counts_row[:, 0:E]
    tot = c1 + counts_row[:, E:TOP_K * E]
    pc = ((tot.astype(_i32) + (TM - 1)) >> 7) << 7
    e_r = lax.broadcasted_iota(_i32, (E, E), 0)
    e_c = lax.broadcasted_iota(_i32, (E, E), 1)
    lt_e = (e_r < e_c).astype(_f32)
    offs = jnp.dot(pc.astype(_f32), lt_e, preferred_element_type=_f32)

    addpack = jnp.concatenate([offs, offs + c1], axis=1)
    dm = (excl + addpack) * ohw
    mask0b = jnp.broadcast_to(mask0.astype(jnp.bool_), dm.shape)
    d0 = jnp.sum(jnp.where(mask0b, dm, 0.0), axis=1, keepdims=True)
    d1 = jnp.sum(jnp.where(mask0b, 0.0, dm), axis=1, keepdims=True)
    d0_ref[...] = d0.astype(_i32)
    d1_ref[...] = d1.astype(_i32)

    starts = (lax.broadcasted_iota(_i32, (NT, E), 0) * TM).astype(_f32)
    te = jnp.sum((offs <= starts).astype(_f32), axis=1, keepdims=True) - 1.0
    te_ref[...] = te.astype(_i32)


def _router_meta(x_flat, router_w):
    return pl.pallas_call(
        _router_meta_body,
        grid=(1,),
        in_specs=[
            pl.BlockSpec((T, H), lambda i: (0, 0)),
            pl.BlockSpec((H, E), lambda i: (0, 0)),
        ],
        out_specs=[
            pl.BlockSpec((T, 1), lambda i: (0, 0)),
            pl.BlockSpec((T, 1), lambda i: (0, 0)),
            pl.BlockSpec((T, 1), lambda i: (0, 0)),
            pl.BlockSpec((T, 1), lambda i: (0, 0)),
            pl.BlockSpec((NT, 1), lambda i: (0, 0)),
        ],
        out_shape=[
            jax.ShapeDtypeStruct((T, 1), _i32),
            jax.ShapeDtypeStruct((T, 1), _i32),
            jax.ShapeDtypeStruct((T, 1), _f32),
            jax.ShapeDtypeStruct((T, 1), _f32),
            jax.ShapeDtypeStruct((NT, 1), _i32),
        ],
    )(x_flat, router_w)


_DC = 64
_PPW = P // NW
_DNCH = _PPW // _DC


def _dispatch_body(x_hbm, dest_hbm, w_hbm, xg_hbm, wdest_hbm,
                   tok0, tok1, dst0, dst1, wv0, wv1, rows0, rows1,
                   sg0, sg1, sr0, sr1, sw0, sw1):
    wid = lax.axis_index("s") * NC + lax.axis_index("c")
    toks = (tok0, tok1)
    dsts = (dst0, dst1)
    wvs = (wv0, wv1)
    rows = (rows0, rows1)
    sg = (sg0, sg1)
    sr = (sr0, sr1)
    sw = (sw0, sw1)

    def stage(ch):
        b = ch & 1
        base = wid * _PPW + ch * _DC
        for g in range(_DC // LANES):
            toks[b][pl.ds(g * LANES, LANES)] = jnp.bitwise_and(
                base + g * LANES + lax.iota(_i32, LANES), T - 1)
        pltpu.sync_copy(dest_hbm.at[pl.ds(base, _DC)], dsts[b])
        pltpu.sync_copy(w_hbm.at[pl.ds(base, _DC)], wvs[b])
        return pltpu.async_copy(x_hbm.at[toks[b]], rows[b], sg[b])

    pend_g = stage(0)
    pend_s = [None, None]
    for ch in range(_DNCH):
        b = ch & 1
        pend_g.wait()
        if ch + 1 < _DNCH:
            if pend_s[1 - b] is not None:
                pend_s[1 - b][0].wait()
                pend_s[1 - b][1].wait()
                pend_s[1 - b] = None
            pend_g = stage(ch + 1)
        pend_s[b] = (
            pltpu.async_copy(rows[b], xg_hbm.at[dsts[b]], sr[b]),
            pltpu.async_copy(wvs[b], wdest_hbm.at[dsts[b]], sw[b]),
        )
    for b in range(2):
        if pend_s[b] is not None:
            pend_s[b][0].wait()
            pend_s[b][1].wait()


@functools.cache
def _dispatch():
    return pl.kernel(
        _dispatch_body,
        out_type=(
            jax.ShapeDtypeStruct((NP, H), _f32),
            jax.ShapeDtypeStruct((NP,), _f32),
        ),
        mesh=plsc.VectorSubcoreMesh(core_axis_name="c", subcore_axis_name="s",
                                    num_cores=NC, num_subcores=NS),
        scratch_types=[
            pltpu.VMEM((_DC,), _i32),
            pltpu.VMEM((_DC,), _i32),
            pltpu.VMEM((_DC,), _i32),
            pltpu.VMEM((_DC,), _i32),
            pltpu.VMEM((_DC,), _f32),
            pltpu.VMEM((_DC,), _f32),
            pltpu.VMEM((_DC, H), _f32),
            pltpu.VMEM((_DC, H), _f32),
            pltpu.SemaphoreType.DMA,
            pltpu.SemaphoreType.DMA,
            pltpu.SemaphoreType.DMA,
            pltpu.SemaphoreType.DMA,
            pltpu.SemaphoreType.DMA,
            pltpu.SemaphoreType.DMA,
        ],
    )


def _gmm_body(te_ref, xg_ref, wg_ref, wu_ref, wd_ref, ws_ref, y_ref):
    xb = xg_ref[...]
    g = jnp.dot(xb, wg_ref[0], preferred_element_type=_f32)
    u = jnp.dot(xb, wu_ref[0], preferred_element_type=_f32)
    h = (g * jax.nn.sigmoid(g)) * u
    y = jnp.dot(h, wd_ref[0], preferred_element_type=_f32)
    y_ref[...] = y * ws_ref[...]


def _gmm(te, xg, w_gate, w_up, w_down, wdest):
    grid_spec = pltpu.PrefetchScalarGridSpec(
        num_scalar_prefetch=1,
        grid=(NT,),
        in_specs=[
            pl.BlockSpec((TM, H), lambda m, te_r: (m, 0)),
            pl.BlockSpec((1, H, F), lambda m, te_r: (te_r[m], 0, 0)),
            pl.BlockSpec((1, H, F), lambda m, te_r: (te_r[m], 0, 0)),
            pl.BlockSpec((1, F, H), lambda m, te_r: (te_r[m], 0, 0)),
            pl.BlockSpec((TM, 1), lambda m, te_r: (m, 0)),
        ],
        out_specs=pl.BlockSpec((TM, H), lambda m, te_r: (m, 0)),
    )
    return pl.pallas_call(
        _gmm_body,
        grid_spec=grid_spec,
        out_shape=jax.ShapeDtypeStruct((NP, H), _f32),
    )(te, xg, w_gate, w_up, w_down, wdest)


_CC = 32
_TPW = T // NW
_CNCH = _TPW // _CC


def _combine_body(y_hbm, dest_hbm, out_hbm,
                  d0a, d0b, d1a, d1b, y0a, y0b, y1a, y1b,
                  sa0, sa1, sb0, sb1, so0, so1):
    wid = lax.axis_index("s") * NC + lax.axis_index("c")
    d0s = (d0a, d0b)
    d1s = (d1a, d1b)
    y0s = (y0a, y0b)
    y1s = (y1a, y1b)
    sas = (sa0, sa1)
    sbs = (sb0, sb1)
    sos = (so0, so1)

    def stage(ch):
        b = ch & 1
        base = wid * _TPW + ch * _CC
        pltpu.sync_copy(dest_hbm.at[pl.ds(base, _CC)], d0s[b])
        pltpu.sync_copy(dest_hbm.at[pl.ds(T + base, _CC)], d1s[b])
        return (pltpu.async_copy(y_hbm.at[d0s[b]], y0s[b], sas[b]),
                pltpu.async_copy(y_hbm.at[d1s[b]], y1s[b], sbs[b]))

    pend_g = stage(0)
    pend_o = [None, None]
    for ch in range(_CNCH):
        b = ch & 1
        base = wid * _TPW + ch * _CC
        pend_g[0].wait()
        pend_g[1].wait()
        if ch + 1 < _CNCH:
            if pend_o[1 - b] is not None:
                pend_o[1 - b].wait()
                pend_o[1 - b] = None
            pend_g = stage(ch + 1)

        y0_v, y1_v = y0s[b], y1s[b]

        def tok(t, _):
            for g in range(H // LANES):
                sl = pl.ds(g * LANES, LANES)
                y0_v[t, sl] = y0_v[t, sl] + y1_v[t, sl]
            return 0

        lax.fori_loop(0, _CC, tok, 0)
        pend_o[b] = pltpu.async_copy(y0_v, out_hbm.at[pl.ds(base, _CC)],
                                     sos[b])
    for b in range(2):
        if pend_o[b] is not None:
            pend_o[b].wait()


@functools.cache
def _combine():
    return pl.kernel(
        _combine_body,
        out_type=jax.ShapeDtypeStruct((T, H), _f32),
        mesh=plsc.VectorSubcoreMesh(core_axis_name="c", subcore_axis_name="s",
                                    num_cores=NC, num_subcores=NS),
        scratch_types=[
            pltpu.VMEM((_CC,), _i32),
            pltpu.VMEM((_CC,), _i32),
            pltpu.VMEM((_CC,), _i32),
            pltpu.VMEM((_CC,), _i32),
            pltpu.VMEM((_CC, H), _f32),
            pltpu.VMEM((_CC, H), _f32),
            pltpu.VMEM((_CC, H), _f32),
            pltpu.VMEM((_CC, H), _f32),
            pltpu.SemaphoreType.DMA,
            pltpu.SemaphoreType.DMA,
            pltpu.SemaphoreType.DMA,
            pltpu.SemaphoreType.DMA,
            pltpu.SemaphoreType.DMA,
            pltpu.SemaphoreType.DMA,
        ],
    )


def kernel(x, router_w, w_gate, w_up, w_down):
    b, s, h = x.shape
    x_flat = x.reshape(-1, h)
    d0, d1, w1, w2, te2 = _router_meta(x_flat, router_w)
    dest = jnp.concatenate([d0.reshape(-1), d1.reshape(-1)])
    wflat = jnp.concatenate([w1.reshape(-1), w2.reshape(-1)])
    te = te2.reshape(-1)
    xg, wdest = _dispatch()(x_flat, dest, wflat)
    y = _gmm(te, xg, w_gate, w_up, w_down, wdest.reshape(NP, 1))
    out = _combine()(y, dest)
    return out.reshape(b, s, h)


# speedup vs baseline: 1.5065x; 1.2036x over previous
import functools

import jax
import jax.numpy as jnp
from jax import lax
from jax.experimental import pallas as pl
from jax.experimental.pallas import tpu as pltpu
from jax.experimental.pallas import tpu_sc as plsc

H = 768
F = 768
E = 64
TOP_K = 2
T = 8192
P = T * TOP_K
TM = 256
TM_SHIFT = TM.bit_length() - 1
NP = ((P + E * (TM - 1) + TM - 1) // TM) * TM
NT = NP // TM
NB = P // TM

NC = 2
NS = 16
NW = NC * NS
LANES = 16

_f32 = jnp.float32
_i32 = jnp.int32


def _router_meta_body(x_ref, rw_ref, d0_ref, d1_ref, w1_ref, w2_ref, te_ref):
    x = x_ref[...]
    logits = jnp.dot(x, rw_ref[...], preferred_element_type=_f32)

    iota_e = lax.broadcasted_iota(_i32, (T, E), 1)
    m1 = jnp.max(logits, axis=1, keepdims=True)
    i1 = jnp.min(jnp.where(logits == m1, iota_e, E), axis=1, keepdims=True)
    oh1 = iota_e == i1
    l2 = jnp.where(oh1, -1e30, logits)
    m2 = jnp.max(l2, axis=1, keepdims=True)
    i2 = jnp.min(jnp.where(l2 == m2, iota_e, E), axis=1, keepdims=True)
    oh2 = iota_e == i2

    ex = jnp.exp(m2 - m1)
    w1_ref[...] = 1.0 / (1.0 + ex)
    w2_ref[...] = ex / (1.0 + ex)

    ohw = jnp.concatenate([oh1.astype(_f32), oh2.astype(_f32)], axis=1)

    a = ohw
    s = 1
    while s < T:
        shifted = jnp.concatenate(
            [jnp.zeros((s, TOP_K * E), _f32), a[0:T - s, :]], axis=0)
        a = a + shifted
        s *= 2
    excl = a - ohw

    counts_row = a[T - 1:T, :]
    lane2 = lax.broadcasted_iota(_i32, (1, TOP_K * E), 1)
    mask0 = (lane2 < E).astype(_f32)
    c1 = counts_row[:, 0:E]
    tot = c1 + counts_row[:, E:TOP_K * E]
    pc = ((tot.astype(_i32) + (TM - 1)) >> TM_SHIFT) << TM_SHIFT
    e_r = lax.broadcasted_iota(_i32, (E, E), 0)
    e_c = lax.broadcasted_iota(_i32, (E, E), 1)
    lt_e = (e_r < e_c).astype(_f32)
    offs = jnp.dot(pc.astype(_f32), lt_e, preferred_element_type=_f32)

    addpack = jnp.concatenate([offs, offs + c1], axis=1)
    dm = (excl + addpack) * ohw
    mask0b = jnp.broadcast_to(mask0.astype(jnp.bool_), dm.shape)
    d0 = jnp.sum(jnp.where(mask0b, dm, 0.0), axis=1, keepdims=True)
    d1 = jnp.sum(jnp.where(mask0b, 0.0, dm), axis=1, keepdims=True)
    d0_ref[...] = d0.astype(_i32)
    d1_ref[...] = d1.astype(_i32)

    starts = (lax.broadcasted_iota(_i32, (NT, E), 0) * TM).astype(_f32)
    te = jnp.sum((offs <= starts).astype(_f32), axis=1, keepdims=True) - 1.0
    te_ref[...] = te.astype(_i32)


def _router_meta(x_flat, router_w):
    return pl.pallas_call(
        _router_meta_body,
        grid=(1,),
        in_specs=[
            pl.BlockSpec((T, H), lambda i: (0, 0)),
            pl.BlockSpec((H, E), lambda i: (0, 0)),
        ],
        out_specs=[
            pl.BlockSpec((T, 1), lambda i: (0, 0)),
            pl.BlockSpec((T, 1), lambda i: (0, 0)),
            pl.BlockSpec((T, 1), lambda i: (0, 0)),
            pl.BlockSpec((T, 1), lambda i: (0, 0)),
            pl.BlockSpec((NT, 1), lambda i: (0, 0)),
        ],
        out_shape=[
            jax.ShapeDtypeStruct((T, 1), _i32),
            jax.ShapeDtypeStruct((T, 1), _i32),
            jax.ShapeDtypeStruct((T, 1), _f32),
            jax.ShapeDtypeStruct((T, 1), _f32),
            jax.ShapeDtypeStruct((NT, 1), _i32),
        ],
    )(x_flat, router_w)


_DC = 64
_PPW = P // NW
_DNCH = _PPW // _DC


def _dispatch_body(x_hbm, dest_hbm, xg_hbm,
                   tok0, tok1, dst0, dst1, rows0, rows1,
                   sg0, sg1, sr0, sr1):
    wid = lax.axis_index("s") * NC + lax.axis_index("c")
    toks = (tok0, tok1)
    dsts = (dst0, dst1)
    rows = (rows0, rows1)
    sg = (sg0, sg1)
    sr = (sr0, sr1)

    def stage(ch):
        b = ch & 1
        base = wid * _PPW + ch * _DC
        for g in range(_DC // LANES):
            toks[b][pl.ds(g * LANES, LANES)] = jnp.bitwise_and(
                base + g * LANES + lax.iota(_i32, LANES), T - 1)
        pltpu.sync_copy(dest_hbm.at[pl.ds(base, _DC)], dsts[b])
        return pltpu.async_copy(x_hbm.at[toks[b]], rows[b], sg[b])

    pend_g = stage(0)
    pend_s = [None, None]
    for ch in range(_DNCH):
        b = ch & 1
        pend_g.wait()
        if ch + 1 < _DNCH:
            if pend_s[1 - b] is not None:
                pend_s[1 - b].wait()
                pend_s[1 - b] = None
            pend_g = stage(ch + 1)
        pend_s[b] = pltpu.async_copy(rows[b], xg_hbm.at[dsts[b]], sr[b])
    for b in range(2):
        if pend_s[b] is not None:
            pend_s[b].wait()


@functools.cache
def _dispatch():
    return pl.kernel(
        _dispatch_body,
        out_type=jax.ShapeDtypeStruct((NP, H), _f32),
        mesh=plsc.VectorSubcoreMesh(core_axis_name="c", subcore_axis_name="s",
                                    num_cores=NC, num_subcores=NS),
        scratch_types=[
            pltpu.VMEM((_DC,), _i32),
            pltpu.VMEM((_DC,), _i32),
            pltpu.VMEM((_DC,), _i32),
            pltpu.VMEM((_DC,), _i32),
            pltpu.VMEM((_DC, H), _f32),
            pltpu.VMEM((_DC, H), _f32),
            pltpu.SemaphoreType.DMA,
            pltpu.SemaphoreType.DMA,
            pltpu.SemaphoreType.DMA,
            pltpu.SemaphoreType.DMA,
        ],
    )


def _gmm_body(te_ref, xg_ref, wg_ref, wu_ref, wd_ref, y_ref):
    xb = xg_ref[...]
    g = jnp.dot(xb, wg_ref[0], preferred_element_type=_f32)
    u = jnp.dot(xb, wu_ref[0], preferred_element_type=_f32)
    h = (g * jax.nn.sigmoid(g)) * u
    y_ref[...] = jnp.dot(h, wd_ref[0], preferred_element_type=_f32)


def _gmm(te, xg, w_gate, w_up, w_down):
    grid_spec = pltpu.PrefetchScalarGridSpec(
        num_scalar_prefetch=1,
        grid=(NT,),
        in_specs=[
            pl.BlockSpec((TM, H), lambda m, te_r: (m, 0)),
            pl.BlockSpec((1, H, F), lambda m, te_r: (te_r[m], 0, 0)),
            pl.BlockSpec((1, H, F), lambda m, te_r: (te_r[m], 0, 0)),
            pl.BlockSpec((1, F, H), lambda m, te_r: (te_r[m], 0, 0)),
        ],
        out_specs=pl.BlockSpec((TM, H), lambda m, te_r: (m, 0)),
    )
    return pl.pallas_call(
        _gmm_body,
        grid_spec=grid_spec,
        out_shape=jax.ShapeDtypeStruct((NP, H), _f32),
    )(te, xg, w_gate, w_up, w_down)


_CC = 32
_TPW = T // NW
_CNCH = _TPW // _CC


def _combine_body(y_hbm, dest_hbm, w_hbm, out_hbm,
                  d0a, d0b, d1a, d1b, w0a, w0b, w1a, w1b,
                  y0a, y0b, y1a, y1b,
                  sa0, sa1, sb0, sb1, so0, so1):
    wid = lax.axis_index("s") * NC + lax.axis_index("c")
    d0s = (d0a, d0b)
    d1s = (d1a, d1b)
    w0v = (w0a, w0b)
    w1v = (w1a, w1b)
    y0s = (y0a, y0b)
    y1s = (y1a, y1b)
    sas = (sa0, sa1)
    sbs = (sb0, sb1)
    sos = (so0, so1)

    def stage(ch):
        b = ch & 1
        base = wid * _TPW + ch * _CC
        pltpu.sync_copy(dest_hbm.at[pl.ds(base, _CC)], d0s[b])
        pltpu.sync_copy(dest_hbm.at[pl.ds(T + base, _CC)], d1s[b])
        pltpu.sync_copy(w_hbm.at[pl.ds(base, _CC)], w0v[b].at[pl.ds(0, _CC)])
        pltpu.sync_copy(w_hbm.at[pl.ds(T + base, _CC)],
                        w1v[b].at[pl.ds(0, _CC)])
        return (pltpu.async_copy(y_hbm.at[d0s[b]], y0s[b], sas[b]),
                pltpu.async_copy(y_hbm.at[d1s[b]], y1s[b], sbs[b]))

    pend_g = stage(0)
    pend_o = [None, None]
    for ch in range(_CNCH):
        b = ch & 1
        base = wid * _TPW + ch * _CC
        pend_g[0].wait()
        pend_g[1].wait()
        if ch + 1 < _CNCH:
            if pend_o[1 - b] is not None:
                pend_o[1 - b].wait()
                pend_o[1 - b] = None
            pend_g = stage(ch + 1)

        y0_v, y1_v = y0s[b], y1s[b]
        w0_v, w1_v = w0v[b], w1v[b]

        def tok(t, _):
            ws0 = jnp.zeros((LANES,), _f32) + w0_v[pl.ds(t, LANES)][0]
            ws1 = jnp.zeros((LANES,), _f32) + w1_v[pl.ds(t, LANES)][0]
            for g in range(H // LANES):
                sl = pl.ds(g * LANES, LANES)
                y0_v[t, sl] = y0_v[t, sl] * ws0 + y1_v[t, sl] * ws1
            return 0

        lax.fori_loop(0, _CC, tok, 0)
        pend_o[b] = pltpu.async_copy(y0_v, out_hbm.at[pl.ds(base, _CC)],
                                     sos[b])
    for b in range(2):
        if pend_o[b] is not None:
            pend_o[b].wait()


@functools.cache
def _combine():
    return pl.kernel(
        _combine_body,
        out_type=jax.ShapeDtypeStruct((T, H), _f32),
        mesh=plsc.VectorSubcoreMesh(core_axis_name="c", subcore_axis_name="s",
                                    num_cores=NC, num_subcores=NS),
        scratch_types=[
            pltpu.VMEM((_CC,), _i32),
            pltpu.VMEM((_CC,), _i32),
            pltpu.VMEM((_CC,), _i32),
            pltpu.VMEM((_CC,), _i32),
            pltpu.VMEM((_CC + LANES,), _f32),
            pltpu.VMEM((_CC + LANES,), _f32),
            pltpu.VMEM((_CC + LANES,), _f32),
            pltpu.VMEM((_CC + LANES,), _f32),
            pltpu.VMEM((_CC, H), _f32),
            pltpu.VMEM((_CC, H), _f32),
            pltpu.VMEM((_CC, H), _f32),
            pltpu.VMEM((_CC, H), _f32),
            pltpu.SemaphoreType.DMA,
            pltpu.SemaphoreType.DMA,
            pltpu.SemaphoreType.DMA,
            pltpu.SemaphoreType.DMA,
            pltpu.SemaphoreType.DMA,
            pltpu.SemaphoreType.DMA,
        ],
    )


def kernel(x, router_w, w_gate, w_up, w_down):
    b, s, h = x.shape
    x_flat = x.reshape(-1, h)
    d0, d1, w1, w2, te2 = _router_meta(x_flat, router_w)
    dest = jnp.concatenate([d0.reshape(-1), d1.reshape(-1)])
    wflat = jnp.concatenate([w1.reshape(-1), w2.reshape(-1)])
    te = te2.reshape(-1)
    xg = _dispatch()(x_flat, dest)
    y = _gmm(te, xg, w_gate, w_up, w_down)
    out = _combine()(y, dest, wflat)
    return out.reshape(b, s, h)
